# baseline (speedup 1.0000x reference)
import functools

import jax
import jax.numpy as jnp
from jax import lax
from jax.experimental import pallas as pl
from jax.experimental.pallas import tpu as pltpu
from jax.experimental.pallas import tpu_sc as plsc

NUM_DEP_TYPES = 53
BATCH = 16
SEQ = 2048
NC, NS = 2, 16
NW = NC * NS
UPB = 2048
CROWS = 16
NCHUNK = 1024 // CROWS
NBUF = 3


def _body(di_hbm, dj_hbm, dt_hbm, emb_hbm, out_hbm,
          bufs_v, di_v, dj_v, vals_v, cid_v, ra_v, dt_v, tab_v, shared_v,
          fill_sem):
    c_ax = lax.axis_index("c")
    s_ax = lax.axis_index("s")
    r = c_ax * NS + s_ax
    b = r // 2
    h = r % 2

    pltpu.sync_copy(di_hbm.at[pl.ds(b * UPB, UPB)], di_v)
    pltpu.sync_copy(dj_hbm.at[pl.ds(b * UPB, UPB)], dj_v)
    pltpu.sync_copy(dt_hbm.at[pl.ds(b * UPB, UPB)], dt_v)
    pltpu.sync_copy(emb_hbm, tab_v)

    row_base = h * 1024
    def precomp(t, carry):
        sl = pl.ds(t * 16, 16)
        i16 = di_v[sl]
        cid_v[sl] = lax.shift_right_logical(i16 - row_base, 4)
        ra_v[sl] = jnp.bitwise_and(i16, CROWS - 1)
        vals_v[sl] = plsc.load_gather(tab_v, [dt_v[sl]])
        return carry
    lax.fori_loop(0, UPB // 16, precomp, 0)

    ones16 = jnp.full((16,), 1.0, jnp.float32)
    for k in range(SEQ // 16):
        bufs_v[0, 0, pl.ds(k * 16, 16)] = ones16
    pltpu.sync_copy(bufs_v.at[0, pl.ds(0, 1)], shared_v.at[pl.ds(s_ax, 1)])
    plsc.subcore_barrier()
    for n in range(NBUF):
        pltpu.sync_copy(shared_v, bufs_v.at[n])

    fills = []
    for c in range(NCHUNK):
        buf = bufs_v.at[c % NBUF]
        if c >= NBUF:
            fills[c - NBUF].wait()

        @plsc.parallel_loop(0, UPB // 16, unroll=4)
        def sweep(t, buf=buf, c=c):
            sl = pl.ds(t * 16, 16)
            cid = cid_v[sl]
            ra = ra_v[sl]
            j = dj_v[sl]
            if c >= NBUF:
                plsc.store_scatter(buf, [ra, j], ones16, mask=cid == c - NBUF)
            plsc.store_scatter(buf, [ra, j], vals_v[sl], mask=cid == c)

        fills.append(
            pltpu.async_copy(
                buf,
                out_hbm.at[pl.ds(b * SEQ + row_base + c * CROWS, CROWS)],
                fill_sem))

    for n in range(NBUF):
        fills[NCHUNK - NBUF + n].wait()


_dep_mask_sc = functools.partial(
    pl.kernel,
    out_type=jax.ShapeDtypeStruct((BATCH * SEQ, SEQ), jnp.float32),
    mesh=plsc.VectorSubcoreMesh(core_axis_name="c", subcore_axis_name="s"),
    compiler_params=pltpu.CompilerParams(needs_layout_passes=False),
    scratch_types=[
        pltpu.VMEM((NBUF, CROWS, SEQ), jnp.float32),
        pltpu.VMEM((UPB,), jnp.int32),
        pltpu.VMEM((UPB,), jnp.int32),
        pltpu.VMEM((UPB,), jnp.float32),
        pltpu.VMEM((UPB,), jnp.int32),
        pltpu.VMEM((UPB,), jnp.int32),
        pltpu.VMEM((UPB,), jnp.int32),
        pltpu.VMEM((64,), jnp.float32),
        pltpu.VMEM_SHARED((CROWS, SEQ), jnp.float32),
        pltpu.SemaphoreType.DMA,
    ],
)(_body)


def kernel(dep_i, dep_j, dep_type, seq_len, dep_embedding):
    del seq_len

    def prep(a):
        return jnp.concatenate([a, a[:, -1:]], axis=1).reshape(-1).astype(jnp.int32)

    di = prep(dep_i)
    dj = prep(dep_j)
    dt = prep(dep_type)
    tab = jnp.pad(dep_embedding.reshape(-1).astype(jnp.float32),
                  (0, 64 - NUM_DEP_TYPES))
    out = _dep_mask_sc(di, dj, dt, tab)
    return out.reshape(BATCH, SEQ, SEQ)
